# Initial kernel scaffold; baseline (speedup 1.0000x reference)
import jax
import jax.numpy as jnp
from jax.experimental import pallas as pl


def kernel(inp_spike_ids, num_inp_spikes, init_state_0, init_state_1, w0, w1, decay_0, decay_1, thresh_0, thresh_1):
    raise NotImplementedError("write your pallas kernel here")


# speedup vs baseline: 2.9460x; 2.9460x over previous
import jax
import jax.numpy as jnp
from jax.experimental import pallas as pl

_SEQ = 512
_BATCH = 8


def _counts(ids, n, width):
    T, B, S = ids.shape
    ids_f = ids.reshape(T * B, S)
    n_f = n.reshape(T * B)
    mask = (jnp.arange(S)[None, :] < n_f[:, None]).astype(jnp.float32)
    rows = jnp.broadcast_to(jnp.arange(T * B)[:, None], (T * B, S))
    out = jnp.zeros((T * B, width), jnp.float32)
    return out.at[rows, ids_f].add(mask)


def _lif_phase(cur, decay, thresh, v0, sparse_out):
    def step(v, cur_t):
        v_new = decay[None, :] * v + cur_t
        spike = v_new > thresh[None, :]
        v_reset = jnp.where(spike, 0.0, v_new)
        scores = jnp.where(spike, v_new, -jnp.inf)
        return v_reset, (v_reset, scores, jnp.sum(spike, axis=1))

    _, (states, scores, nsp) = jax.lax.scan(step, v0, cur)
    T, B, N = scores.shape
    _, top_ids = jax.lax.top_k(scores.reshape(T * B, N), sparse_out)
    n_out = jnp.minimum(nsp, sparse_out).astype(jnp.int32)
    valid = jnp.arange(sparse_out)[None, :] < n_out.reshape(T * B)[:, None]
    out_ids = jnp.where(valid, top_ids, 0).astype(jnp.int32).reshape(T, B, sparse_out)
    return out_ids, n_out, states


def kernel(inp_spike_ids, num_inp_spikes, init_state_0, init_state_1, w0, w1, decay_0, decay_1, thresh_0, thresh_1):
    T, B = _SEQ, _BATCH
    d0, d1, d2 = w0.shape[1], w0.shape[0], w1.shape[0]
    n0 = num_inp_spikes[..., 0]

    c0 = _counts(inp_spike_ids, n0, d0)
    cur0 = jax.lax.dot_general(
        c0, w0, (((1,), (1,)), ((), ())), precision=jax.lax.Precision.HIGHEST
    ).reshape(T, B, d1)
    ids1, n1, s1 = _lif_phase(cur0, decay_0, thresh_0, init_state_0, 128)

    c1 = _counts(ids1, n1, d1)
    cur1 = jax.lax.dot_general(
        c1, w1, (((1,), (1,)), ((), ())), precision=jax.lax.Precision.HIGHEST
    ).reshape(T, B, d2)
    ids2, n2, s2 = _lif_phase(cur1, decay_1, thresh_1, init_state_1, 64)

    num1 = jnp.stack([n1, jnp.zeros_like(n1)], axis=-1)
    num2 = jnp.stack([n2, jnp.zeros_like(n2)], axis=-1)
    return (ids1, ids2, num1, num2, s1, s2)


# speedup vs baseline: 4.4654x; 1.5158x over previous
import functools

import jax
import jax.numpy as jnp
from jax.experimental import pallas as pl
from jax.experimental.pallas import tpu as pltpu

_SEQ = 512
_BATCH = 8
_TCHUNK = 32
_NEG = float("-inf")


def _counts(ids, n, width):
    T, B, S = ids.shape
    ids_f = ids.reshape(T * B, S)
    n_f = n.reshape(T * B)
    mask = (jnp.arange(S)[None, :] < n_f[:, None]).astype(jnp.float32)
    rows = jnp.broadcast_to(jnp.arange(T * B)[:, None], (T * B, S))
    out = jnp.zeros((T * B, width), jnp.float32)
    return out.at[rows, ids_f].add(mask)


def _mm_scan_body(counts_ref, wt_ref, decay_ref, thresh_ref, v0_ref,
                  states_ref, scores_ref, v_ref, cur_ref):
    step = pl.program_id(0)

    @pl.when(step == 0)
    def _():
        v_ref[...] = v0_ref[...]

    cur = jnp.dot(counts_ref[...], wt_ref[...],
                  precision=jax.lax.Precision.HIGHEST,
                  preferred_element_type=jnp.float32)
    cur_ref[...] = cur.reshape(_TCHUNK, _BATCH, wt_ref.shape[1])
    decay = decay_ref[...]
    thresh = thresh_ref[...]

    def body(i, v):
        v_new = decay * v + cur_ref[i]
        spike = v_new > thresh
        v_reset = jnp.where(spike, 0.0, v_new)
        states_ref[i] = v_reset
        scores_ref[i] = jnp.where(spike, v_new, _NEG)
        return v_reset

    v_ref[...] = jax.lax.fori_loop(0, _TCHUNK, body, v_ref[...])


def _mm_scan(counts, wt, decay, thresh, v0):
    K, N = wt.shape
    T, B = _SEQ, _BATCH
    grid = (T // _TCHUNK,)
    out_shape = [
        jax.ShapeDtypeStruct((T, B, N), jnp.float32),
        jax.ShapeDtypeStruct((T, B, N), jnp.float32),
    ]
    out_specs = [
        pl.BlockSpec((_TCHUNK, B, N), lambda i: (i, 0, 0)),
        pl.BlockSpec((_TCHUNK, B, N), lambda i: (i, 0, 0)),
    ]
    in_specs = [
        pl.BlockSpec((_TCHUNK * B, K), lambda i: (i, 0)),
        pl.BlockSpec((K, N), lambda i: (0, 0)),
        pl.BlockSpec((1, N), lambda i: (0, 0)),
        pl.BlockSpec((1, N), lambda i: (0, 0)),
        pl.BlockSpec((B, N), lambda i: (0, 0)),
    ]
    states, scores = pl.pallas_call(
        _mm_scan_body,
        grid=grid,
        in_specs=in_specs,
        out_specs=out_specs,
        out_shape=out_shape,
        scratch_shapes=[pltpu.VMEM((B, N), jnp.float32),
                        pltpu.VMEM((_TCHUNK, B, N), jnp.float32)],
    )(counts, wt, decay.reshape(1, N), thresh.reshape(1, N), v0)
    return states, scores


def _topk_phase(scores, sparse_out):
    T, B, N = scores.shape
    nsp = jnp.sum(scores > _NEG, axis=-1)
    _, top_ids = jax.lax.top_k(scores.reshape(T * B, N), sparse_out)
    n_out = jnp.minimum(nsp, sparse_out).astype(jnp.int32)
    valid = jnp.arange(sparse_out)[None, :] < n_out.reshape(T * B)[:, None]
    out_ids = jnp.where(valid, top_ids, 0).astype(jnp.int32).reshape(T, B, sparse_out)
    return out_ids, n_out


def kernel(inp_spike_ids, num_inp_spikes, init_state_0, init_state_1, w0, w1, decay_0, decay_1, thresh_0, thresh_1):
    d0, d1, d2 = w0.shape[1], w0.shape[0], w1.shape[0]
    n0 = num_inp_spikes[..., 0]

    c0 = _counts(inp_spike_ids, n0, d0)
    s1, sc1 = _mm_scan(c0, jnp.transpose(w0), decay_0, thresh_0, init_state_0)
    ids1, n1 = _topk_phase(sc1, 128)

    c1 = _counts(ids1, n1, d1)
    s2, sc2 = _mm_scan(c1, jnp.transpose(w1), decay_1, thresh_1, init_state_1)
    ids2, n2 = _topk_phase(sc2, 64)

    num1 = jnp.stack([n1, jnp.zeros_like(n1)], axis=-1)
    num2 = jnp.stack([n2, jnp.zeros_like(n2)], axis=-1)
    return (ids1, ids2, num1, num2, s1, s2)


# speedup vs baseline: 11.9485x; 2.6758x over previous
import functools

import jax
import jax.numpy as jnp
from jax import lax
from jax.experimental import pallas as pl
from jax.experimental.pallas import tpu as pltpu
from jax.experimental.pallas import tpu_sc as plsc

_SEQ = 512
_BATCH = 8
_NW = 32
_GROUP = 16
_TCHUNK = 32
_NEG = float("-inf")


@functools.cache
def _counts_kernel(rows, S, width):
    per_w = rows // _NW
    groups = per_w // _GROUP
    mesh = plsc.VectorSubcoreMesh(core_axis_name="c", subcore_axis_name="s")

    @functools.partial(
        pl.kernel, mesh=mesh,
        out_type=jax.ShapeDtypeStruct((rows * width,), jnp.float32),
        compiler_params=pltpu.CompilerParams(needs_layout_passes=False),
        scratch_types=[
            pltpu.VMEM((_GROUP * S,), jnp.int32),
            pltpu.VMEM((_GROUP,), jnp.int32),
            pltpu.VMEM((_GROUP * width,), jnp.float32),
        ],
    )
    def k(ids_hbm, n_hbm, zeros_hbm, out_hbm, ids_v, n_v, cnt_v):
        wid = lax.axis_index("s") * 2 + lax.axis_index("c")
        lanes = jnp.arange(_GROUP, dtype=jnp.int32)
        ones = jnp.ones((_GROUP,), jnp.float32)

        def group_body(g, _):
            row0 = wid * per_w + g * _GROUP
            pltpu.sync_copy(zeros_hbm, cnt_v)
            pltpu.sync_copy(ids_hbm.at[pl.ds(row0 * S, _GROUP * S)], ids_v)
            pltpu.sync_copy(n_hbm.at[pl.ds(row0, _GROUP)], n_v)
            nvec = n_v[...]

            def slot_body(j, _):
                idx = plsc.load_gather(ids_v, [lanes * S + j])
                m = jnp.full((_GROUP,), j, jnp.int32) < nvec
                plsc.addupdate_scatter(cnt_v, [lanes * width + idx], ones, mask=m)
                return 0

            lax.fori_loop(0, S, slot_body, 0)
            pltpu.sync_copy(cnt_v, out_hbm.at[pl.ds(row0 * width, _GROUP * width)])
            return 0

        lax.fori_loop(0, groups, group_body, 0)

    return k


def _counts(ids, n, width):
    T, B, S = ids.shape
    rows = T * B
    zeros = jnp.zeros((_GROUP * width,), jnp.float32)
    out = _counts_kernel(rows, S, width)(ids.reshape(rows * S), n.reshape(rows), zeros)
    return out.reshape(rows, width)


def _mm_scan_body(counts_ref, wt_ref, decay_ref, thresh_ref, v0_ref,
                  states_ref, scores_ref, v_ref, cur_ref):
    step = pl.program_id(0)

    @pl.when(step == 0)
    def _():
        v_ref[...] = v0_ref[...]

    cur = jnp.dot(counts_ref[...], wt_ref[...],
                  precision=jax.lax.Precision.HIGHEST,
                  preferred_element_type=jnp.float32)
    cur_ref[...] = cur.reshape(_TCHUNK, _BATCH, wt_ref.shape[1])
    decay = decay_ref[...]
    thresh = thresh_ref[...]

    def body(i, v):
        v_new = decay * v + cur_ref[i]
        spike = v_new > thresh
        v_reset = jnp.where(spike, 0.0, v_new)
        states_ref[i] = v_reset
        scores_ref[i] = jnp.where(spike, v_new, _NEG)
        return v_reset

    v_ref[...] = jax.lax.fori_loop(0, _TCHUNK, body, v_ref[...])


def _mm_scan(counts, wt, decay, thresh, v0):
    K, N = wt.shape
    T, B = _SEQ, _BATCH
    grid = (T // _TCHUNK,)
    out_shape = [
        jax.ShapeDtypeStruct((T, B, N), jnp.float32),
        jax.ShapeDtypeStruct((T, B, N), jnp.float32),
    ]
    out_specs = [
        pl.BlockSpec((_TCHUNK, B, N), lambda i: (i, 0, 0)),
        pl.BlockSpec((_TCHUNK, B, N), lambda i: (i, 0, 0)),
    ]
    in_specs = [
        pl.BlockSpec((_TCHUNK * B, K), lambda i: (i, 0)),
        pl.BlockSpec((K, N), lambda i: (0, 0)),
        pl.BlockSpec((1, N), lambda i: (0, 0)),
        pl.BlockSpec((1, N), lambda i: (0, 0)),
        pl.BlockSpec((B, N), lambda i: (0, 0)),
    ]
    states, scores = pl.pallas_call(
        _mm_scan_body,
        grid=grid,
        in_specs=in_specs,
        out_specs=out_specs,
        out_shape=out_shape,
        scratch_shapes=[pltpu.VMEM((B, N), jnp.float32),
                        pltpu.VMEM((_TCHUNK, B, N), jnp.float32)],
    )(counts, wt, decay.reshape(1, N), thresh.reshape(1, N), v0)
    return states, scores


def _topk_phase(scores, sparse_out):
    T, B, N = scores.shape
    nsp = jnp.sum(scores > _NEG, axis=-1)
    _, top_ids = jax.lax.top_k(scores.reshape(T * B, N), sparse_out)
    n_out = jnp.minimum(nsp, sparse_out).astype(jnp.int32)
    valid = jnp.arange(sparse_out)[None, :] < n_out.reshape(T * B)[:, None]
    out_ids = jnp.where(valid, top_ids, 0).astype(jnp.int32).reshape(T, B, sparse_out)
    return out_ids, n_out


def kernel(inp_spike_ids, num_inp_spikes, init_state_0, init_state_1, w0, w1, decay_0, decay_1, thresh_0, thresh_1):
    d0, d1, d2 = w0.shape[1], w0.shape[0], w1.shape[0]
    n0 = num_inp_spikes[..., 0]

    c0 = _counts(inp_spike_ids, n0, d0)
    s1, sc1 = _mm_scan(c0, jnp.transpose(w0), decay_0, thresh_0, init_state_0)
    ids1, n1 = _topk_phase(sc1, 128)

    c1 = _counts(ids1, n1, d1)
    s2, sc2 = _mm_scan(c1, jnp.transpose(w1), decay_1, thresh_1, init_state_1)
    ids2, n2 = _topk_phase(sc2, 64)

    num1 = jnp.stack([n1, jnp.zeros_like(n1)], axis=-1)
    num2 = jnp.stack([n2, jnp.zeros_like(n2)], axis=-1)
    return (ids1, ids2, num1, num2, s1, s2)


# speedup vs baseline: 31.1380x; 2.6060x over previous
import functools

import jax
import jax.numpy as jnp
from jax import lax
from jax.experimental import pallas as pl
from jax.experimental.pallas import tpu as pltpu
from jax.experimental.pallas import tpu_sc as plsc

_SEQ = 512
_BATCH = 8
_NW = 32
_GROUP = 16
_TCHUNK = 32
_NEG = float("-inf")


@functools.cache
def _counts_kernel(rows, S, width):
    per_w = rows // _NW
    groups = per_w // _GROUP
    mesh = plsc.VectorSubcoreMesh(core_axis_name="c", subcore_axis_name="s")

    @functools.partial(
        pl.kernel, mesh=mesh,
        out_type=jax.ShapeDtypeStruct((rows * width,), jnp.float32),
        compiler_params=pltpu.CompilerParams(needs_layout_passes=False),
        scratch_types=[
            pltpu.VMEM((_GROUP * S,), jnp.int32),
            pltpu.VMEM((_GROUP,), jnp.int32),
            pltpu.VMEM((_GROUP * width,), jnp.float32),
        ],
    )
    def k(ids_hbm, n_hbm, zeros_hbm, out_hbm, ids_v, n_v, cnt_v):
        wid = lax.axis_index("s") * 2 + lax.axis_index("c")
        lanes = jnp.arange(_GROUP, dtype=jnp.int32)
        ones = jnp.ones((_GROUP,), jnp.float32)

        def group_body(g, _):
            row0 = wid * per_w + g * _GROUP
            pltpu.sync_copy(zeros_hbm, cnt_v)
            pltpu.sync_copy(ids_hbm.at[pl.ds(row0 * S, _GROUP * S)], ids_v)
            pltpu.sync_copy(n_hbm.at[pl.ds(row0, _GROUP)], n_v)
            nvec = n_v[...]

            def slot_body(j, _):
                idx = plsc.load_gather(ids_v, [lanes * S + j])
                m = jnp.full((_GROUP,), j, jnp.int32) < nvec
                plsc.addupdate_scatter(cnt_v, [lanes * width + idx], ones, mask=m)
                return 0

            lax.fori_loop(0, S, slot_body, 0)
            pltpu.sync_copy(cnt_v, out_hbm.at[pl.ds(row0 * width, _GROUP * width)])
            return 0

        lax.fori_loop(0, groups, group_body, 0)

    return k


def _counts(ids, n, width):
    T, B, S = ids.shape
    rows = T * B
    zeros = jnp.zeros((_GROUP * width,), jnp.float32)
    out = _counts_kernel(rows, S, width)(ids.reshape(rows * S), n.reshape(rows), zeros)
    return out.reshape(rows, width)


def _mm_scan_body(counts_ref, wt_ref, decay_ref, thresh_ref, v0_ref,
                  states_ref, scores_ref, v_ref, cur_ref):
    step = pl.program_id(0)

    @pl.when(step == 0)
    def _():
        v_ref[...] = v0_ref[...]

    cur = jnp.dot(counts_ref[...], wt_ref[...],
                  precision=jax.lax.Precision.HIGHEST,
                  preferred_element_type=jnp.float32)
    cur_ref[...] = cur.reshape(_TCHUNK, _BATCH, wt_ref.shape[1])
    decay = decay_ref[...]
    thresh = thresh_ref[...]

    def body(i, v):
        v_new = decay * v + cur_ref[i]
        spike = v_new > thresh
        v_reset = jnp.where(spike, 0.0, v_new)
        states_ref[i] = v_reset
        scores_ref[i] = jnp.where(spike, v_new, _NEG)
        return v_reset

    v_ref[...] = jax.lax.fori_loop(0, _TCHUNK, body, v_ref[...])


def _mm_scan(counts, wt, decay, thresh, v0):
    K, N = wt.shape
    T, B = _SEQ, _BATCH
    grid = (T // _TCHUNK,)
    out_shape = [
        jax.ShapeDtypeStruct((T, B, N), jnp.float32),
        jax.ShapeDtypeStruct((T, B, N), jnp.float32),
    ]
    out_specs = [
        pl.BlockSpec((_TCHUNK, B, N), lambda i: (i, 0, 0)),
        pl.BlockSpec((_TCHUNK, B, N), lambda i: (i, 0, 0)),
    ]
    in_specs = [
        pl.BlockSpec((_TCHUNK * B, K), lambda i: (i, 0)),
        pl.BlockSpec((K, N), lambda i: (0, 0)),
        pl.BlockSpec((1, N), lambda i: (0, 0)),
        pl.BlockSpec((1, N), lambda i: (0, 0)),
        pl.BlockSpec((B, N), lambda i: (0, 0)),
    ]
    states, scores = pl.pallas_call(
        _mm_scan_body,
        grid=grid,
        in_specs=in_specs,
        out_specs=out_specs,
        out_shape=out_shape,
        scratch_shapes=[pltpu.VMEM((B, N), jnp.float32),
                        pltpu.VMEM((_TCHUNK, B, N), jnp.float32)],
    )(counts, wt, decay.reshape(1, N), thresh.reshape(1, N), v0)
    return states, scores


def _pairstep(ka, va, kb, vb):
    cmp = ka >= kb
    return (jnp.where(cmp, ka, kb), jnp.where(cmp, va, vb),
            jnp.where(cmp, kb, ka), jnp.where(cmp, vb, va))


def _bitonic_merge(ak, av, bk, bv):
    m = len(ak)
    lk = list(ak) + [lax.rev(k, (0,)) for k in reversed(bk)]
    lv = list(av) + [lax.rev(v, (0,)) for v in reversed(bv)]
    n = 2 * m
    b = m
    while b >= 1:
        for blk in range(0, n, 2 * b):
            for i in range(blk, blk + b):
                hi_k, hi_v, lo_k, lo_v = _pairstep(lk[i], lv[i], lk[i + b], lv[i + b])
                lk[i], lv[i] = hi_k, hi_v
                lk[i + b], lv[i + b] = lo_k, lo_v
        b //= 2
    out = [plsc.sort_key_val(k, v, descending=True) for k, v in zip(lk, lv)]
    return [k for k, _ in out], [v for _, v in out]


def _sort_desc(ks, vs):
    runs = []
    for k, v in zip(ks, vs):
        sk, sv = plsc.sort_key_val(k, v, descending=True)
        runs.append(([sk], [sv]))
    while len(runs) > 1:
        nxt = []
        for i in range(0, len(runs), 2):
            ak, av = runs[i]
            bk, bv = runs[i + 1]
            nxt.append(tuple(_bitonic_merge(ak, av, bk, bv)))
        runs = nxt
    return runs[0]


@functools.cache
def _topk_kernel(rows, N, K, emit_counts):
    KCH = K // 16
    NCH = N // 16
    per_w = rows // _NW
    groups = per_w // _GROUP
    mesh = plsc.VectorSubcoreMesh(core_axis_name="c", subcore_axis_name="s")

    out_type = [
        jax.ShapeDtypeStruct((rows * K,), jnp.int32),
        jax.ShapeDtypeStruct((rows,), jnp.int32),
    ]
    scratch = [
        pltpu.VMEM((_GROUP * N,), jnp.float32),
        pltpu.VMEM((_GROUP * K,), jnp.int32),
        pltpu.VMEM((_GROUP,), jnp.int32),
        pltpu.VMEM((N + 16,), jnp.float32),
        pltpu.VMEM((N + 16,), jnp.int32),
    ]
    if emit_counts:
        out_type.append(jax.ShapeDtypeStruct((rows * N,), jnp.float32))
        scratch.append(pltpu.VMEM((_GROUP * N,), jnp.float32))

    @functools.partial(
        pl.kernel, mesh=mesh, out_type=out_type,
        compiler_params=pltpu.CompilerParams(needs_layout_passes=False),
        scratch_types=scratch,
    )
    def k(scores_hbm, zeros_hbm, ids_hbm, n_hbm, *rest):
        if emit_counts:
            counts_hbm, scores_v, ids_v, n_v, vals_v, idxs_v, cnt_v = rest
        else:
            scores_v, ids_v, n_v, vals_v, idxs_v = rest
        wid = lax.axis_index("s") * 2 + lax.axis_index("c")
        lanes = jnp.arange(_GROUP, dtype=jnp.int32)
        neg = jnp.full((16,), _NEG, jnp.float32)
        ones = jnp.ones((16,), jnp.float32)

        def group_body(g, _):
            row0 = wid * per_w + g * _GROUP
            pltpu.sync_copy(scores_hbm.at[pl.ds(row0 * N, _GROUP * N)], scores_v)
            if emit_counts:
                pltpu.sync_copy(zeros_hbm, cnt_v)

            def row_body(r, n_acc):
                base = r * N
                for j in range(KCH):
                    vals_v[pl.ds(j * 16, 16)] = neg

                def cbody(c, cur):
                    s = scores_v[pl.ds(base + c * 16, 16)]
                    m = s > -3e38
                    plsc.store_compressed(vals_v.at[pl.ds(cur, 16)], s, mask=m)
                    plsc.store_compressed(
                        idxs_v.at[pl.ds(cur, 16)], lanes + c * 16, mask=m)
                    return cur + jnp.sum(m.astype(jnp.int32))

                n_raw = lax.fori_loop(0, NCH, cbody, 0)
                vals_v[pl.ds(n_raw, 16)] = neg

                tk = [vals_v[pl.ds(j * 16, 16)] for j in range(KCH)]
                tv = [idxs_v[pl.ds(j * 16, 16)] for j in range(KCH)]
                tk, tv = _sort_desc(tk, tv)

                def xbody(c, carry):
                    ck, cv = carry
                    ck = list(ck)
                    cv = list(cv)
                    ek = vals_v[pl.ds(c * 16, 16)]
                    ev = idxs_v[pl.ds(c * 16, 16)]
                    ek, ev = plsc.sort_key_val(ek, ev, descending=True)
                    rk = lax.rev(ek, (0,))
                    rv = lax.rev(ev, (0,))
                    hi_k, hi_v, _, _ = _pairstep(ck[-1], cv[-1], rk, rv)
                    ck[-1] = hi_k
                    cv[-1] = hi_v
                    sk, sv = _sort_desc(ck, cv)
                    return tuple(sk), tuple(sv)

                nc = (n_raw + 15) // 16
                tk, tv = lax.fori_loop(KCH, nc, xbody, (tuple(tk), tuple(tv)))

                kk = jnp.minimum(n_raw, K)
                for j in range(KCH):
                    m = (lanes + j * 16) < kk
                    ids_v[pl.ds(r * K + j * 16, 16)] = jnp.where(m, tv[j], 0)
                    if emit_counts:
                        plsc.addupdate_scatter(cnt_v, [r * N + tv[j]], ones, mask=m)
                return jnp.where(lanes == r, kk, n_acc)

            n_acc = lax.fori_loop(0, _GROUP, row_body,
                                  jnp.zeros((_GROUP,), jnp.int32))
            n_v[...] = n_acc
            pltpu.sync_copy(ids_v, ids_hbm.at[pl.ds(row0 * K, _GROUP * K)])
            pltpu.sync_copy(n_v, n_hbm.at[pl.ds(row0, _GROUP)])
            if emit_counts:
                pltpu.sync_copy(cnt_v, counts_hbm.at[pl.ds(row0 * N, _GROUP * N)])
            return 0

        lax.fori_loop(0, groups, group_body, 0)

    return k


def _topk_phase(scores, sparse_out, emit_counts):
    T, B, N = scores.shape
    rows = T * B
    zeros = jnp.zeros((_GROUP * N,), jnp.float32)
    outs = _topk_kernel(rows, N, sparse_out, emit_counts)(
        scores.reshape(rows * N), zeros)
    ids = outs[0].reshape(T, B, sparse_out)
    n_out = outs[1].reshape(T, B)
    if emit_counts:
        return ids, n_out, outs[2].reshape(rows, N)
    return ids, n_out


def kernel(inp_spike_ids, num_inp_spikes, init_state_0, init_state_1, w0, w1, decay_0, decay_1, thresh_0, thresh_1):
    d0, d1, d2 = w0.shape[1], w0.shape[0], w1.shape[0]
    n0 = num_inp_spikes[..., 0]

    c0 = _counts(inp_spike_ids, n0, d0)
    s1, sc1 = _mm_scan(c0, jnp.transpose(w0), decay_0, thresh_0, init_state_0)
    ids1, n1, c1 = _topk_phase(sc1, 128, True)

    s2, sc2 = _mm_scan(c1, jnp.transpose(w1), decay_1, thresh_1, init_state_1)
    ids2, n2 = _topk_phase(sc2, 64, False)

    num1 = jnp.stack([n1, jnp.zeros_like(n1)], axis=-1)
    num2 = jnp.stack([n2, jnp.zeros_like(n2)], axis=-1)
    return (ids1, ids2, num1, num2, s1, s2)


# speedup vs baseline: 31.6060x; 1.0150x over previous
import functools

import jax
import jax.numpy as jnp
from jax import lax
from jax.experimental import pallas as pl
from jax.experimental.pallas import tpu as pltpu
from jax.experimental.pallas import tpu_sc as plsc

_SEQ = 512
_BATCH = 8
_NW = 32
_GROUP = 16
_TCHUNK = 64
_NEG = float("-inf")


@functools.cache
def _counts_kernel(rows, S, width):
    per_w = rows // _NW
    groups = per_w // _GROUP
    mesh = plsc.VectorSubcoreMesh(core_axis_name="c", subcore_axis_name="s")

    @functools.partial(
        pl.kernel, mesh=mesh,
        out_type=jax.ShapeDtypeStruct((rows * width,), jnp.float32),
        compiler_params=pltpu.CompilerParams(needs_layout_passes=False),
        scratch_types=[
            pltpu.VMEM((_GROUP * S,), jnp.int32),
            pltpu.VMEM((_GROUP,), jnp.int32),
            pltpu.VMEM((_GROUP * width,), jnp.float32),
        ],
    )
    def k(ids_hbm, n_hbm, zeros_hbm, out_hbm, ids_v, n_v, cnt_v):
        wid = lax.axis_index("s") * 2 + lax.axis_index("c")
        lanes = jnp.arange(_GROUP, dtype=jnp.int32)
        ones = jnp.ones((_GROUP,), jnp.float32)

        def group_body(g, _):
            row0 = wid * per_w + g * _GROUP
            pltpu.sync_copy(zeros_hbm, cnt_v)
            pltpu.sync_copy(ids_hbm.at[pl.ds(row0 * S, _GROUP * S)], ids_v)
            pltpu.sync_copy(n_hbm.at[pl.ds(row0, _GROUP)], n_v)
            nvec = n_v[...]

            def slot_body(j, _):
                idx = plsc.load_gather(ids_v, [lanes * S + j])
                m = jnp.full((_GROUP,), j, jnp.int32) < nvec
                plsc.addupdate_scatter(cnt_v, [lanes * width + idx], ones, mask=m)
                return 0

            lax.fori_loop(0, S, slot_body, 0)
            pltpu.sync_copy(cnt_v, out_hbm.at[pl.ds(row0 * width, _GROUP * width)])
            return 0

        lax.fori_loop(0, groups, group_body, 0)

    return k


def _counts(ids, n, width):
    T, B, S = ids.shape
    rows = T * B
    zeros = jnp.zeros((_GROUP * width,), jnp.float32)
    out = _counts_kernel(rows, S, width)(ids.reshape(rows * S), n.reshape(rows), zeros)
    return out.reshape(rows, width)


def _mm_scan_body(counts_ref, wt_ref, decay_ref, thresh_ref, v0_ref,
                  states_ref, scores_ref, v_ref, cur_ref):
    step = pl.program_id(0)

    @pl.when(step == 0)
    def _():
        v_ref[...] = v0_ref[...]

    cur = jnp.dot(counts_ref[...], wt_ref[...],
                  precision=jax.lax.Precision.HIGHEST,
                  preferred_element_type=jnp.float32)
    cur_ref[...] = cur.reshape(_TCHUNK, _BATCH, wt_ref.shape[1])
    decay = decay_ref[...]
    thresh = thresh_ref[...]

    def body(i, v):
        v_new = decay * v + cur_ref[i]
        spike = v_new > thresh
        v_reset = jnp.where(spike, 0.0, v_new)
        states_ref[i] = v_reset
        scores_ref[i] = jnp.where(spike, v_new, _NEG)
        return v_reset

    v_ref[...] = jax.lax.fori_loop(0, _TCHUNK, body, v_ref[...])


def _mm_scan(counts, wt, decay, thresh, v0):
    K, N = wt.shape
    T, B = _SEQ, _BATCH
    grid = (T // _TCHUNK,)
    out_shape = [
        jax.ShapeDtypeStruct((T, B, N), jnp.float32),
        jax.ShapeDtypeStruct((T, B, N), jnp.float32),
    ]
    out_specs = [
        pl.BlockSpec((_TCHUNK, B, N), lambda i: (i, 0, 0)),
        pl.BlockSpec((_TCHUNK, B, N), lambda i: (i, 0, 0)),
    ]
    in_specs = [
        pl.BlockSpec((_TCHUNK * B, K), lambda i: (i, 0)),
        pl.BlockSpec((K, N), lambda i: (0, 0)),
        pl.BlockSpec((1, N), lambda i: (0, 0)),
        pl.BlockSpec((1, N), lambda i: (0, 0)),
        pl.BlockSpec((B, N), lambda i: (0, 0)),
    ]
    states, scores = pl.pallas_call(
        _mm_scan_body,
        grid=grid,
        in_specs=in_specs,
        out_specs=out_specs,
        out_shape=out_shape,
        scratch_shapes=[pltpu.VMEM((B, N), jnp.float32),
                        pltpu.VMEM((_TCHUNK, B, N), jnp.float32)],
    )(counts, wt, decay.reshape(1, N), thresh.reshape(1, N), v0)
    return states, scores


def _pairstep(ka, va, kb, vb):
    cmp = ka >= kb
    return (jnp.where(cmp, ka, kb), jnp.where(cmp, va, vb),
            jnp.where(cmp, kb, ka), jnp.where(cmp, vb, va))


def _bitonic_merge(ak, av, bk, bv):
    m = len(ak)
    lk = list(ak) + [lax.rev(k, (0,)) for k in reversed(bk)]
    lv = list(av) + [lax.rev(v, (0,)) for v in reversed(bv)]
    n = 2 * m
    b = m
    while b >= 1:
        for blk in range(0, n, 2 * b):
            for i in range(blk, blk + b):
                hi_k, hi_v, lo_k, lo_v = _pairstep(lk[i], lv[i], lk[i + b], lv[i + b])
                lk[i], lv[i] = hi_k, hi_v
                lk[i + b], lv[i + b] = lo_k, lo_v
        b //= 2
    out = [plsc.sort_key_val(k, v, descending=True) for k, v in zip(lk, lv)]
    return [k for k, _ in out], [v for _, v in out]


def _sort_desc(ks, vs):
    runs = []
    for k, v in zip(ks, vs):
        sk, sv = plsc.sort_key_val(k, v, descending=True)
        runs.append(([sk], [sv]))
    while len(runs) > 1:
        nxt = []
        for i in range(0, len(runs), 2):
            ak, av = runs[i]
            bk, bv = runs[i + 1]
            nxt.append(tuple(_bitonic_merge(ak, av, bk, bv)))
        runs = nxt
    return runs[0]


@functools.cache
def _topk_kernel(rows, N, K, emit_counts):
    KCH = K // 16
    NCH = N // 16
    per_w = rows // _NW
    groups = per_w // _GROUP
    mesh = plsc.VectorSubcoreMesh(core_axis_name="c", subcore_axis_name="s")

    out_type = [
        jax.ShapeDtypeStruct((rows * K,), jnp.int32),
        jax.ShapeDtypeStruct((rows,), jnp.int32),
    ]
    scratch = [
        pltpu.VMEM((_GROUP * N,), jnp.float32),
        pltpu.VMEM((_GROUP * K,), jnp.int32),
        pltpu.VMEM((_GROUP,), jnp.int32),
        pltpu.VMEM((N + 16,), jnp.float32),
        pltpu.VMEM((N + 16,), jnp.int32),
    ]
    if emit_counts:
        out_type.append(jax.ShapeDtypeStruct((rows * N,), jnp.float32))
        scratch.append(pltpu.VMEM((_GROUP * N,), jnp.float32))

    @functools.partial(
        pl.kernel, mesh=mesh, out_type=out_type,
        compiler_params=pltpu.CompilerParams(needs_layout_passes=False),
        scratch_types=scratch,
    )
    def k(scores_hbm, zeros_hbm, ids_hbm, n_hbm, *rest):
        if emit_counts:
            counts_hbm, scores_v, ids_v, n_v, vals_v, idxs_v, cnt_v = rest
        else:
            scores_v, ids_v, n_v, vals_v, idxs_v = rest
        wid = lax.axis_index("s") * 2 + lax.axis_index("c")
        lanes = jnp.arange(_GROUP, dtype=jnp.int32)
        neg = jnp.full((16,), _NEG, jnp.float32)
        ones = jnp.ones((16,), jnp.float32)

        def group_body(g, _):
            row0 = wid * per_w + g * _GROUP
            pltpu.sync_copy(scores_hbm.at[pl.ds(row0 * N, _GROUP * N)], scores_v)
            if emit_counts:
                pltpu.sync_copy(zeros_hbm, cnt_v)

            def row_body(r, n_acc):
                base = r * N
                for j in range(KCH):
                    vals_v[pl.ds(j * 16, 16)] = neg

                def cbody(c, cur):
                    s = scores_v[pl.ds(base + c * 16, 16)]
                    m = s > -3e38
                    plsc.store_compressed(vals_v.at[pl.ds(cur, 16)], s, mask=m)
                    plsc.store_compressed(
                        idxs_v.at[pl.ds(cur, 16)], lanes + c * 16, mask=m)
                    return cur + jnp.sum(m.astype(jnp.int32))

                n_raw = lax.fori_loop(0, NCH, cbody, 0)
                vals_v[pl.ds(n_raw, 16)] = neg

                tk = [vals_v[pl.ds(j * 16, 16)] for j in range(KCH)]
                tv = [idxs_v[pl.ds(j * 16, 16)] for j in range(KCH)]
                tk, tv = _sort_desc(tk, tv)

                def xbody(c, carry):
                    ck, cv = carry
                    ck = list(ck)
                    cv = list(cv)
                    ek = vals_v[pl.ds(c * 16, 16)]
                    ev = idxs_v[pl.ds(c * 16, 16)]
                    ek, ev = plsc.sort_key_val(ek, ev, descending=True)
                    rk = lax.rev(ek, (0,))
                    rv = lax.rev(ev, (0,))
                    hi_k, hi_v, _, _ = _pairstep(ck[-1], cv[-1], rk, rv)
                    ck[-1] = hi_k
                    cv[-1] = hi_v
                    sk, sv = _sort_desc(ck, cv)
                    return tuple(sk), tuple(sv)

                nc = (n_raw + 15) // 16
                tk, tv = lax.fori_loop(KCH, nc, xbody, (tuple(tk), tuple(tv)))

                kk = jnp.minimum(n_raw, K)
                for j in range(KCH):
                    m = (lanes + j * 16) < kk
                    ids_v[pl.ds(r * K + j * 16, 16)] = jnp.where(m, tv[j], 0)
                    if emit_counts:
                        plsc.addupdate_scatter(cnt_v, [r * N + tv[j]], ones, mask=m)
                return jnp.where(lanes == r, kk, n_acc)

            n_acc = lax.fori_loop(0, _GROUP, row_body,
                                  jnp.zeros((_GROUP,), jnp.int32))
            n_v[...] = n_acc
            pltpu.sync_copy(ids_v, ids_hbm.at[pl.ds(row0 * K, _GROUP * K)])
            pltpu.sync_copy(n_v, n_hbm.at[pl.ds(row0, _GROUP)])
            if emit_counts:
                pltpu.sync_copy(cnt_v, counts_hbm.at[pl.ds(row0 * N, _GROUP * N)])
            return 0

        lax.fori_loop(0, groups, group_body, 0)

    return k


def _topk_phase(scores, sparse_out, emit_counts):
    T, B, N = scores.shape
    rows = T * B
    zeros = jnp.zeros((_GROUP * N,), jnp.float32)
    outs = _topk_kernel(rows, N, sparse_out, emit_counts)(
        scores.reshape(rows * N), zeros)
    ids = outs[0].reshape(T, B, sparse_out)
    n_out = outs[1].reshape(T, B)
    if emit_counts:
        return ids, n_out, outs[2].reshape(rows, N)
    return ids, n_out


def kernel(inp_spike_ids, num_inp_spikes, init_state_0, init_state_1, w0, w1, decay_0, decay_1, thresh_0, thresh_1):
    d0, d1, d2 = w0.shape[1], w0.shape[0], w1.shape[0]
    n0 = num_inp_spikes[..., 0]

    c0 = _counts(inp_spike_ids, n0, d0)
    s1, sc1 = _mm_scan(c0, jnp.transpose(w0), decay_0, thresh_0, init_state_0)
    ids1, n1, c1 = _topk_phase(sc1, 128, True)

    s2, sc2 = _mm_scan(c1, jnp.transpose(w1), decay_1, thresh_1, init_state_1)
    ids2, n2 = _topk_phase(sc2, 64, False)

    num1 = jnp.stack([n1, jnp.zeros_like(n1)], axis=-1)
    num2 = jnp.stack([n2, jnp.zeros_like(n2)], axis=-1)
    return (ids1, ids2, num1, num2, s1, s2)


# speedup vs baseline: 34.6434x; 1.0961x over previous
import functools

import jax
import jax.numpy as jnp
from jax import lax
from jax.experimental import pallas as pl
from jax.experimental.pallas import tpu as pltpu
from jax.experimental.pallas import tpu_sc as plsc

_SEQ = 512
_BATCH = 8
_NW = 32
_GROUP = 16
_TCHUNK = 64
_NEG = float("-inf")


@functools.cache
def _counts_kernel(rows, S, width):
    per_w = rows // _NW
    groups = per_w // _GROUP
    mesh = plsc.VectorSubcoreMesh(core_axis_name="c", subcore_axis_name="s")

    GS = _GROUP * S
    GW = _GROUP * width

    @functools.partial(
        pl.kernel, mesh=mesh,
        out_type=jax.ShapeDtypeStruct((rows * width,), jnp.float32),
        compiler_params=pltpu.CompilerParams(needs_layout_passes=False),
        scratch_types=[
            pltpu.VMEM((2 * GS,), jnp.int32),
            pltpu.VMEM((2 * _GROUP,), jnp.int32),
            pltpu.VMEM((2 * GW,), jnp.float32),
            pltpu.SemaphoreType.DMA,
            pltpu.SemaphoreType.DMA,
        ],
    )
    def k(ids_hbm, n_hbm, zeros_hbm, out_hbm, ids_v, n_v, cnt_v, sem_in, sem_out):
        wid = lax.axis_index("s") * 2 + lax.axis_index("c")
        lanes = jnp.arange(_GROUP, dtype=jnp.int32)
        ones = jnp.ones((_GROUP,), jnp.float32)

        def in_desc(g, par):
            row0 = wid * per_w + g * _GROUP
            return [
                pltpu.make_async_copy(ids_hbm.at[pl.ds(row0 * S, GS)],
                                      ids_v.at[pl.ds(par * GS, GS)], sem_in),
                pltpu.make_async_copy(n_hbm.at[pl.ds(row0, _GROUP)],
                                      n_v.at[pl.ds(par * _GROUP, _GROUP)], sem_in),
                pltpu.make_async_copy(zeros_hbm,
                                      cnt_v.at[pl.ds(par * GW, GW)], sem_in),
            ]

        def out_desc(g, par):
            row0 = wid * per_w + g * _GROUP
            return [
                pltpu.make_async_copy(cnt_v.at[pl.ds(par * GW, GW)],
                                      out_hbm.at[pl.ds(row0 * width, GW)], sem_out),
            ]

        for d in in_desc(0, 0):
            d.start()

        def group_body(g, _):
            par = lax.rem(g, 2)

            @pl.when(g > 0)
            def _():
                for d in out_desc(g - 1, 1 - par):
                    d.wait()

            for d in in_desc(g, par):
                d.wait()

            @pl.when(g + 1 < groups)
            def _():
                for d in in_desc(g + 1, 1 - par):
                    d.start()

            nvec = n_v[pl.ds(par * _GROUP, _GROUP)]

            def slot_body(j, _):
                idx = plsc.load_gather(ids_v, [par * GS + lanes * S + j])
                m = jnp.full((_GROUP,), j, jnp.int32) < nvec
                plsc.addupdate_scatter(
                    cnt_v, [par * GW + lanes * width + idx], ones, mask=m)
                return 0

            lax.fori_loop(0, S, slot_body, 0)
            for d in out_desc(g, par):
                d.start()
            return 0

        lax.fori_loop(0, groups, group_body, 0)
        for d in out_desc(groups - 1, (groups - 1) % 2):
            d.wait()

    return k


def _counts(ids, n, width):
    T, B, S = ids.shape
    rows = T * B
    zeros = jnp.zeros((_GROUP * width,), jnp.float32)
    out = _counts_kernel(rows, S, width)(ids.reshape(rows * S), n.reshape(rows), zeros)
    return out.reshape(rows, width)


def _mm_scan_body(counts_ref, wt_ref, decay_ref, thresh_ref, v0_ref,
                  states_ref, scores_ref, v_ref, cur_ref):
    step = pl.program_id(0)

    @pl.when(step == 0)
    def _():
        v_ref[...] = v0_ref[...]

    cur = jnp.dot(counts_ref[...], wt_ref[...],
                  precision=jax.lax.Precision.HIGHEST,
                  preferred_element_type=jnp.float32)
    cur_ref[...] = cur.reshape(_TCHUNK, _BATCH, wt_ref.shape[1])
    decay = decay_ref[...]
    thresh = thresh_ref[...]

    def body(i, v):
        v_new = decay * v + cur_ref[i]
        spike = v_new > thresh
        v_reset = jnp.where(spike, 0.0, v_new)
        states_ref[i] = v_reset
        scores_ref[i] = jnp.where(spike, v_new, _NEG)
        return v_reset

    v_ref[...] = jax.lax.fori_loop(0, _TCHUNK, body, v_ref[...])


def _mm_scan(counts, wt, decay, thresh, v0):
    K, N = wt.shape
    T, B = _SEQ, _BATCH
    grid = (T // _TCHUNK,)
    out_shape = [
        jax.ShapeDtypeStruct((T, B, N), jnp.float32),
        jax.ShapeDtypeStruct((T, B, N), jnp.float32),
    ]
    out_specs = [
        pl.BlockSpec((_TCHUNK, B, N), lambda i: (i, 0, 0)),
        pl.BlockSpec((_TCHUNK, B, N), lambda i: (i, 0, 0)),
    ]
    in_specs = [
        pl.BlockSpec((_TCHUNK * B, K), lambda i: (i, 0)),
        pl.BlockSpec((K, N), lambda i: (0, 0)),
        pl.BlockSpec((1, N), lambda i: (0, 0)),
        pl.BlockSpec((1, N), lambda i: (0, 0)),
        pl.BlockSpec((B, N), lambda i: (0, 0)),
    ]
    states, scores = pl.pallas_call(
        _mm_scan_body,
        grid=grid,
        in_specs=in_specs,
        out_specs=out_specs,
        out_shape=out_shape,
        scratch_shapes=[pltpu.VMEM((B, N), jnp.float32),
                        pltpu.VMEM((_TCHUNK, B, N), jnp.float32)],
    )(counts, wt, decay.reshape(1, N), thresh.reshape(1, N), v0)
    return states, scores


def _pairstep(ka, va, kb, vb):
    cmp = ka >= kb
    return (jnp.where(cmp, ka, kb), jnp.where(cmp, va, vb),
            jnp.where(cmp, kb, ka), jnp.where(cmp, vb, va))


def _bitonic_merge(ak, av, bk, bv):
    m = len(ak)
    lk = list(ak) + [lax.rev(k, (0,)) for k in reversed(bk)]
    lv = list(av) + [lax.rev(v, (0,)) for v in reversed(bv)]
    n = 2 * m
    b = m
    while b >= 1:
        for blk in range(0, n, 2 * b):
            for i in range(blk, blk + b):
                hi_k, hi_v, lo_k, lo_v = _pairstep(lk[i], lv[i], lk[i + b], lv[i + b])
                lk[i], lv[i] = hi_k, hi_v
                lk[i + b], lv[i + b] = lo_k, lo_v
        b //= 2
    out = [plsc.sort_key_val(k, v, descending=True) for k, v in zip(lk, lv)]
    return [k for k, _ in out], [v for _, v in out]


def _sort_desc(ks, vs):
    runs = []
    for k, v in zip(ks, vs):
        sk, sv = plsc.sort_key_val(k, v, descending=True)
        runs.append(([sk], [sv]))
    while len(runs) > 1:
        nxt = []
        for i in range(0, len(runs), 2):
            ak, av = runs[i]
            bk, bv = runs[i + 1]
            nxt.append(tuple(_bitonic_merge(ak, av, bk, bv)))
        runs = nxt
    return runs[0]


@functools.cache
def _topk_kernel(rows, N, K, emit_counts):
    KCH = K // 16
    NCH = N // 16
    per_w = rows // _NW
    groups = per_w // _GROUP
    mesh = plsc.VectorSubcoreMesh(core_axis_name="c", subcore_axis_name="s")

    GN = _GROUP * N
    GK = _GROUP * K
    out_type = [
        jax.ShapeDtypeStruct((rows * K,), jnp.int32),
        jax.ShapeDtypeStruct((rows,), jnp.int32),
    ]
    scratch = [
        pltpu.VMEM((2 * GN,), jnp.float32),
        pltpu.VMEM((2 * GK,), jnp.int32),
        pltpu.VMEM((2 * _GROUP,), jnp.int32),
        pltpu.VMEM((N + 16,), jnp.float32),
        pltpu.VMEM((N + 16,), jnp.int32),
        pltpu.SemaphoreType.DMA,
        pltpu.SemaphoreType.DMA,
    ]
    if emit_counts:
        out_type.append(jax.ShapeDtypeStruct((rows * N,), jnp.float32))
        scratch.append(pltpu.VMEM((2 * GN,), jnp.float32))

    @functools.partial(
        pl.kernel, mesh=mesh, out_type=out_type,
        compiler_params=pltpu.CompilerParams(needs_layout_passes=False),
        scratch_types=scratch,
    )
    def k(scores_hbm, zeros_hbm, ids_hbm, n_hbm, *rest):
        if emit_counts:
            counts_hbm, scores_v, ids_v, n_v, vals_v, idxs_v, sem_in, sem_out, cnt_v = rest
        else:
            scores_v, ids_v, n_v, vals_v, idxs_v, sem_in, sem_out = rest
        wid = lax.axis_index("s") * 2 + lax.axis_index("c")
        lanes = jnp.arange(_GROUP, dtype=jnp.int32)
        neg = jnp.full((16,), _NEG, jnp.float32)
        ones = jnp.ones((16,), jnp.float32)

        def in_desc(g, par):
            row0 = wid * per_w + g * _GROUP
            ds = [pltpu.make_async_copy(scores_hbm.at[pl.ds(row0 * N, GN)],
                                        scores_v.at[pl.ds(par * GN, GN)], sem_in)]
            if emit_counts:
                ds.append(pltpu.make_async_copy(
                    zeros_hbm, cnt_v.at[pl.ds(par * GN, GN)], sem_in))
            return ds

        def out_desc(g, par):
            row0 = wid * per_w + g * _GROUP
            ds = [
                pltpu.make_async_copy(ids_v.at[pl.ds(par * GK, GK)],
                                      ids_hbm.at[pl.ds(row0 * K, GK)], sem_out),
                pltpu.make_async_copy(n_v.at[pl.ds(par * _GROUP, _GROUP)],
                                      n_hbm.at[pl.ds(row0, _GROUP)], sem_out),
            ]
            if emit_counts:
                ds.append(pltpu.make_async_copy(
                    cnt_v.at[pl.ds(par * GN, GN)],
                    counts_hbm.at[pl.ds(row0 * N, GN)], sem_out))
            return ds

        for d in in_desc(0, 0):
            d.start()

        def group_body(g, _):
            par = lax.rem(g, 2)

            @pl.when(g > 0)
            def _():
                for d in out_desc(g - 1, 1 - par):
                    d.wait()

            for d in in_desc(g, par):
                d.wait()

            @pl.when(g + 1 < groups)
            def _():
                for d in in_desc(g + 1, 1 - par):
                    d.start()

            def row_body(r, n_acc):
                base = par * GN + r * N
                for j in range(KCH):
                    vals_v[pl.ds(j * 16, 16)] = neg

                def cbody(c, cur):
                    s = scores_v[pl.ds(base + c * 16, 16)]
                    m = s > -3e38
                    plsc.store_compressed(vals_v.at[pl.ds(cur, 16)], s, mask=m)
                    plsc.store_compressed(
                        idxs_v.at[pl.ds(cur, 16)], lanes + c * 16, mask=m)
                    return cur + jnp.sum(m.astype(jnp.int32))

                n_raw = lax.fori_loop(0, NCH, cbody, 0)
                vals_v[pl.ds(n_raw, 16)] = neg

                tk = [vals_v[pl.ds(j * 16, 16)] for j in range(KCH)]
                tv = [idxs_v[pl.ds(j * 16, 16)] for j in range(KCH)]
                tk, tv = _sort_desc(tk, tv)

                def xbody(c, carry):
                    ck, cv = carry
                    ck = list(ck)
                    cv = list(cv)
                    ek = vals_v[pl.ds(c * 16, 16)]
                    ev = idxs_v[pl.ds(c * 16, 16)]
                    ek, ev = plsc.sort_key_val(ek, ev, descending=True)
                    rk = lax.rev(ek, (0,))
                    rv = lax.rev(ev, (0,))
                    hi_k, hi_v, _, _ = _pairstep(ck[-1], cv[-1], rk, rv)
                    ck[-1] = hi_k
                    cv[-1] = hi_v
                    sk, sv = _sort_desc(ck, cv)
                    return tuple(sk), tuple(sv)

                nc = (n_raw + 15) // 16
                tk, tv = lax.fori_loop(KCH, nc, xbody, (tuple(tk), tuple(tv)))

                kk = jnp.minimum(n_raw, K)
                for j in range(KCH):
                    m = (lanes + j * 16) < kk
                    ids_v[pl.ds(par * GK + r * K + j * 16, 16)] = jnp.where(m, tv[j], 0)
                    if emit_counts:
                        plsc.addupdate_scatter(
                            cnt_v, [par * GN + r * N + tv[j]], ones, mask=m)
                return jnp.where(lanes == r, kk, n_acc)

            n_acc = lax.fori_loop(0, _GROUP, row_body,
                                  jnp.zeros((_GROUP,), jnp.int32))
            n_v[pl.ds(par * _GROUP, _GROUP)] = n_acc
            for d in out_desc(g, par):
                d.start()
            return 0

        lax.fori_loop(0, groups, group_body, 0)
        for d in out_desc(groups - 1, (groups - 1) % 2):
            d.wait()

    return k


def _topk_phase(scores, sparse_out, emit_counts):
    T, B, N = scores.shape
    rows = T * B
    zeros = jnp.zeros((_GROUP * N,), jnp.float32)
    outs = _topk_kernel(rows, N, sparse_out, emit_counts)(
        scores.reshape(rows * N), zeros)
    ids = outs[0].reshape(T, B, sparse_out)
    n_out = outs[1].reshape(T, B)
    if emit_counts:
        return ids, n_out, outs[2].reshape(rows, N)
    return ids, n_out


def kernel(inp_spike_ids, num_inp_spikes, init_state_0, init_state_1, w0, w1, decay_0, decay_1, thresh_0, thresh_1):
    d0, d1, d2 = w0.shape[1], w0.shape[0], w1.shape[0]
    n0 = num_inp_spikes[..., 0]

    c0 = _counts(inp_spike_ids, n0, d0)
    s1, sc1 = _mm_scan(c0, jnp.transpose(w0), decay_0, thresh_0, init_state_0)
    ids1, n1, c1 = _topk_phase(sc1, 128, True)

    s2, sc2 = _mm_scan(c1, jnp.transpose(w1), decay_1, thresh_1, init_state_1)
    ids2, n2 = _topk_phase(sc2, 64, False)

    num1 = jnp.stack([n1, jnp.zeros_like(n1)], axis=-1)
    num2 = jnp.stack([n2, jnp.zeros_like(n2)], axis=-1)
    return (ids1, ids2, num1, num2, s1, s2)


# speedup vs baseline: 37.9231x; 1.0947x over previous
import functools

import jax
import jax.numpy as jnp
from jax import lax
from jax.experimental import pallas as pl
from jax.experimental.pallas import tpu as pltpu
from jax.experimental.pallas import tpu_sc as plsc

_SEQ = 512
_BATCH = 8
_NW = 32
_GROUP = 16
_TCHUNK = 64
_NEG = float("-inf")


@functools.cache
def _counts_kernel(rows, S, width):
    per_w = rows // _NW
    groups = per_w // _GROUP
    mesh = plsc.VectorSubcoreMesh(core_axis_name="c", subcore_axis_name="s")

    GS = _GROUP * S
    GW = _GROUP * width

    @functools.partial(
        pl.kernel, mesh=mesh,
        out_type=jax.ShapeDtypeStruct((rows * width,), jnp.float32),
        compiler_params=pltpu.CompilerParams(needs_layout_passes=False),
        scratch_types=[
            pltpu.VMEM((2 * GS,), jnp.int32),
            pltpu.VMEM((2 * _GROUP,), jnp.int32),
            pltpu.VMEM((2 * GW,), jnp.float32),
            pltpu.SemaphoreType.DMA,
            pltpu.SemaphoreType.DMA,
        ],
    )
    def k(ids_hbm, n_hbm, zeros_hbm, out_hbm, ids_v, n_v, cnt_v, sem_in, sem_out):
        wid = lax.axis_index("s") * 2 + lax.axis_index("c")
        lanes = jnp.arange(_GROUP, dtype=jnp.int32)
        ones = jnp.ones((_GROUP,), jnp.float32)

        def in_desc(g, par):
            row0 = wid * per_w + g * _GROUP
            return [
                pltpu.make_async_copy(ids_hbm.at[pl.ds(row0 * S, GS)],
                                      ids_v.at[pl.ds(par * GS, GS)], sem_in),
                pltpu.make_async_copy(n_hbm.at[pl.ds(row0, _GROUP)],
                                      n_v.at[pl.ds(par * _GROUP, _GROUP)], sem_in),
                pltpu.make_async_copy(zeros_hbm,
                                      cnt_v.at[pl.ds(par * GW, GW)], sem_in),
            ]

        def out_desc(g, par):
            row0 = wid * per_w + g * _GROUP
            return [
                pltpu.make_async_copy(cnt_v.at[pl.ds(par * GW, GW)],
                                      out_hbm.at[pl.ds(row0 * width, GW)], sem_out),
            ]

        for d in in_desc(0, 0):
            d.start()

        def group_body(g, _):
            par = lax.rem(g, 2)

            @pl.when(g > 0)
            def _():
                for d in out_desc(g - 1, 1 - par):
                    d.wait()

            for d in in_desc(g, par):
                d.wait()

            @pl.when(g + 1 < groups)
            def _():
                for d in in_desc(g + 1, 1 - par):
                    d.start()

            nvec = n_v[pl.ds(par * _GROUP, _GROUP)]
            nmax = jnp.max(nvec)

            def slot_body(h, _):
                for u in range(2):
                    j = h * 2 + u
                    idx = plsc.load_gather(ids_v, [par * GS + lanes * S + j])
                    m = jnp.full((_GROUP,), j, jnp.int32) < nvec
                    plsc.addupdate_scatter(
                        cnt_v, [par * GW + lanes * width + idx], ones, mask=m)
                return 0

            lax.fori_loop(0, (nmax + 1) // 2, slot_body, 0)
            for d in out_desc(g, par):
                d.start()
            return 0

        lax.fori_loop(0, groups, group_body, 0)
        for d in out_desc(groups - 1, (groups - 1) % 2):
            d.wait()

    return k


def _counts(ids, n, width):
    T, B, S = ids.shape
    rows = T * B
    zeros = jnp.zeros((_GROUP * width,), jnp.float32)
    out = _counts_kernel(rows, S, width)(ids.reshape(rows * S), n.reshape(rows), zeros)
    return out.reshape(rows, width)


def _mm_scan_body(counts_ref, wt1_ref, wt2_ref, wt3_ref, decay_ref, thresh_ref,
                  v0_ref, states_ref, scores_ref, v_ref, cur_ref):
    step = pl.program_id(0)

    @pl.when(step == 0)
    def _():
        v_ref[...] = v0_ref[...]

    cb = counts_ref[...].astype(jnp.bfloat16)
    cur = (jnp.dot(cb, wt1_ref[...], preferred_element_type=jnp.float32)
           + jnp.dot(cb, wt2_ref[...], preferred_element_type=jnp.float32)
           + jnp.dot(cb, wt3_ref[...], preferred_element_type=jnp.float32))
    cur_ref[...] = cur.reshape(_TCHUNK, _BATCH, wt1_ref.shape[1])
    decay = decay_ref[...]
    thresh = thresh_ref[...]

    def body(i, v):
        v_new = decay * v + cur_ref[i]
        spike = v_new > thresh
        v_reset = jnp.where(spike, 0.0, v_new)
        states_ref[i] = v_reset
        scores_ref[i] = jnp.where(spike, v_new, _NEG)
        return v_reset

    v_ref[...] = jax.lax.fori_loop(0, _TCHUNK, body, v_ref[...])


def _mm_scan(counts, wt, decay, thresh, v0):
    K, N = wt.shape
    T, B = _SEQ, _BATCH
    wt1 = wt.astype(jnp.bfloat16)
    wt2 = (wt - wt1.astype(jnp.float32)).astype(jnp.bfloat16)
    wt3 = (wt - wt1.astype(jnp.float32) - wt2.astype(jnp.float32)).astype(jnp.bfloat16)
    grid = (T // _TCHUNK,)
    out_shape = [
        jax.ShapeDtypeStruct((T, B, N), jnp.float32),
        jax.ShapeDtypeStruct((T, B, N), jnp.float32),
    ]
    out_specs = [
        pl.BlockSpec((_TCHUNK, B, N), lambda i: (i, 0, 0)),
        pl.BlockSpec((_TCHUNK, B, N), lambda i: (i, 0, 0)),
    ]
    in_specs = [
        pl.BlockSpec((_TCHUNK * B, K), lambda i: (i, 0)),
        pl.BlockSpec((K, N), lambda i: (0, 0)),
        pl.BlockSpec((K, N), lambda i: (0, 0)),
        pl.BlockSpec((K, N), lambda i: (0, 0)),
        pl.BlockSpec((1, N), lambda i: (0, 0)),
        pl.BlockSpec((1, N), lambda i: (0, 0)),
        pl.BlockSpec((B, N), lambda i: (0, 0)),
    ]
    states, scores = pl.pallas_call(
        _mm_scan_body,
        grid=grid,
        in_specs=in_specs,
        out_specs=out_specs,
        out_shape=out_shape,
        scratch_shapes=[pltpu.VMEM((B, N), jnp.float32),
                        pltpu.VMEM((_TCHUNK, B, N), jnp.float32)],
    )(counts, wt1, wt2, wt3, decay.reshape(1, N), thresh.reshape(1, N), v0)
    return states, scores


def _pairstep(ka, va, kb, vb):
    cmp = ka >= kb
    return (jnp.where(cmp, ka, kb), jnp.where(cmp, va, vb),
            jnp.where(cmp, kb, ka), jnp.where(cmp, vb, va))


def _bitonic_merge(ak, av, bk, bv):
    m = len(ak)
    lk = list(ak) + [lax.rev(k, (0,)) for k in reversed(bk)]
    lv = list(av) + [lax.rev(v, (0,)) for v in reversed(bv)]
    n = 2 * m
    b = m
    while b >= 1:
        for blk in range(0, n, 2 * b):
            for i in range(blk, blk + b):
                hi_k, hi_v, lo_k, lo_v = _pairstep(lk[i], lv[i], lk[i + b], lv[i + b])
                lk[i], lv[i] = hi_k, hi_v
                lk[i + b], lv[i + b] = lo_k, lo_v
        b //= 2
    out = [plsc.sort_key_val(k, v, descending=True) for k, v in zip(lk, lv)]
    return [k for k, _ in out], [v for _, v in out]


def _sort_desc(ks, vs):
    runs = []
    for k, v in zip(ks, vs):
        sk, sv = plsc.sort_key_val(k, v, descending=True)
        runs.append(([sk], [sv]))
    while len(runs) > 1:
        nxt = []
        for i in range(0, len(runs), 2):
            ak, av = runs[i]
            bk, bv = runs[i + 1]
            nxt.append(tuple(_bitonic_merge(ak, av, bk, bv)))
        runs = nxt
    return runs[0]


@functools.cache
def _topk_kernel(rows, N, K, emit_counts):
    KCH = K // 16
    NCH = N // 16
    per_w = rows // _NW
    groups = per_w // _GROUP
    mesh = plsc.VectorSubcoreMesh(core_axis_name="c", subcore_axis_name="s")

    GN = _GROUP * N
    GK = _GROUP * K
    out_type = [
        jax.ShapeDtypeStruct((rows * K,), jnp.int32),
        jax.ShapeDtypeStruct((rows,), jnp.int32),
    ]
    scratch = [
        pltpu.VMEM((2 * GN,), jnp.float32),
        pltpu.VMEM((2 * GK,), jnp.int32),
        pltpu.VMEM((2 * _GROUP,), jnp.int32),
        pltpu.VMEM((N + 16,), jnp.float32),
        pltpu.VMEM((N + 16,), jnp.int32),
        pltpu.SemaphoreType.DMA,
        pltpu.SemaphoreType.DMA,
    ]
    if emit_counts:
        out_type.append(jax.ShapeDtypeStruct((rows * N,), jnp.float32))
        scratch.append(pltpu.VMEM((2 * GN,), jnp.float32))

    @functools.partial(
        pl.kernel, mesh=mesh, out_type=out_type,
        compiler_params=pltpu.CompilerParams(needs_layout_passes=False),
        scratch_types=scratch,
    )
    def k(scores_hbm, zeros_hbm, ids_hbm, n_hbm, *rest):
        if emit_counts:
            counts_hbm, scores_v, ids_v, n_v, vals_v, idxs_v, sem_in, sem_out, cnt_v = rest
        else:
            scores_v, ids_v, n_v, vals_v, idxs_v, sem_in, sem_out = rest
        wid = lax.axis_index("s") * 2 + lax.axis_index("c")
        lanes = jnp.arange(_GROUP, dtype=jnp.int32)
        neg = jnp.full((16,), _NEG, jnp.float32)
        ones = jnp.ones((16,), jnp.float32)

        def in_desc(g, par):
            row0 = wid * per_w + g * _GROUP
            ds = [pltpu.make_async_copy(scores_hbm.at[pl.ds(row0 * N, GN)],
                                        scores_v.at[pl.ds(par * GN, GN)], sem_in)]
            if emit_counts:
                ds.append(pltpu.make_async_copy(
                    zeros_hbm, cnt_v.at[pl.ds(par * GN, GN)], sem_in))
            return ds

        def out_desc(g, par):
            row0 = wid * per_w + g * _GROUP
            ds = [
                pltpu.make_async_copy(ids_v.at[pl.ds(par * GK, GK)],
                                      ids_hbm.at[pl.ds(row0 * K, GK)], sem_out),
                pltpu.make_async_copy(n_v.at[pl.ds(par * _GROUP, _GROUP)],
                                      n_hbm.at[pl.ds(row0, _GROUP)], sem_out),
            ]
            if emit_counts:
                ds.append(pltpu.make_async_copy(
                    cnt_v.at[pl.ds(par * GN, GN)],
                    counts_hbm.at[pl.ds(row0 * N, GN)], sem_out))
            return ds

        for d in in_desc(0, 0):
            d.start()

        def group_body(g, _):
            par = lax.rem(g, 2)

            @pl.when(g > 0)
            def _():
                for d in out_desc(g - 1, 1 - par):
                    d.wait()

            for d in in_desc(g, par):
                d.wait()

            @pl.when(g + 1 < groups)
            def _():
                for d in in_desc(g + 1, 1 - par):
                    d.start()

            def row_body(r, n_acc):
                base = par * GN + r * N
                for j in range(KCH):
                    vals_v[pl.ds(j * 16, 16)] = neg

                def cbody(c, cur):
                    s = scores_v[pl.ds(base + c * 16, 16)]
                    m = s > -3e38
                    plsc.store_compressed(vals_v.at[pl.ds(cur, 16)], s, mask=m)
                    plsc.store_compressed(
                        idxs_v.at[pl.ds(cur, 16)], lanes + c * 16, mask=m)
                    return cur + jnp.sum(m.astype(jnp.int32))

                n_raw = lax.fori_loop(0, NCH, cbody, 0)
                vals_v[pl.ds(n_raw, 16)] = neg

                tk = [vals_v[pl.ds(j * 16, 16)] for j in range(KCH)]
                tv = [idxs_v[pl.ds(j * 16, 16)] for j in range(KCH)]
                tk, tv = _sort_desc(tk, tv)

                def xbody(c, carry):
                    ck, cv = carry
                    ck = list(ck)
                    cv = list(cv)
                    ek = vals_v[pl.ds(c * 16, 16)]
                    ev = idxs_v[pl.ds(c * 16, 16)]
                    ek, ev = plsc.sort_key_val(ek, ev, descending=True)
                    rk = lax.rev(ek, (0,))
                    rv = lax.rev(ev, (0,))
                    hi_k, hi_v, _, _ = _pairstep(ck[-1], cv[-1], rk, rv)
                    ck[-1] = hi_k
                    cv[-1] = hi_v
                    sk, sv = _sort_desc(ck, cv)
                    return tuple(sk), tuple(sv)

                nc = (n_raw + 15) // 16
                tk, tv = lax.fori_loop(KCH, nc, xbody, (tuple(tk), tuple(tv)))

                kk = jnp.minimum(n_raw, K)
                for j in range(KCH):
                    m = (lanes + j * 16) < kk
                    ids_v[pl.ds(par * GK + r * K + j * 16, 16)] = jnp.where(m, tv[j], 0)
                    if emit_counts:
                        plsc.addupdate_scatter(
                            cnt_v, [par * GN + r * N + tv[j]], ones, mask=m)
                return jnp.where(lanes == r, kk, n_acc)

            n_acc = lax.fori_loop(0, _GROUP, row_body,
                                  jnp.zeros((_GROUP,), jnp.int32))
            n_v[pl.ds(par * _GROUP, _GROUP)] = n_acc
            for d in out_desc(g, par):
                d.start()
            return 0

        lax.fori_loop(0, groups, group_body, 0)
        for d in out_desc(groups - 1, (groups - 1) % 2):
            d.wait()

    return k


def _topk_phase(scores, sparse_out, emit_counts):
    T, B, N = scores.shape
    rows = T * B
    zeros = jnp.zeros((_GROUP * N,), jnp.float32)
    outs = _topk_kernel(rows, N, sparse_out, emit_counts)(
        scores.reshape(rows * N), zeros)
    ids = outs[0].reshape(T, B, sparse_out)
    n_out = outs[1].reshape(T, B)
    if emit_counts:
        return ids, n_out, outs[2].reshape(rows, N)
    return ids, n_out


def kernel(inp_spike_ids, num_inp_spikes, init_state_0, init_state_1, w0, w1, decay_0, decay_1, thresh_0, thresh_1):
    d0, d1, d2 = w0.shape[1], w0.shape[0], w1.shape[0]
    n0 = num_inp_spikes[..., 0]

    c0 = _counts(inp_spike_ids, n0, d0)
    s1, sc1 = _mm_scan(c0, jnp.transpose(w0), decay_0, thresh_0, init_state_0)
    ids1, n1, c1 = _topk_phase(sc1, 128, True)

    s2, sc2 = _mm_scan(c1, jnp.transpose(w1), decay_1, thresh_1, init_state_1)
    ids2, n2 = _topk_phase(sc2, 64, False)

    num1 = jnp.stack([n1, jnp.zeros_like(n1)], axis=-1)
    num2 = jnp.stack([n2, jnp.zeros_like(n2)], axis=-1)
    return (ids1, ids2, num1, num2, s1, s2)
